# baseline (speedup 1.0000x reference)
import functools

import jax
import jax.numpy as jnp
from jax import lax
from jax.experimental import pallas as pl
from jax.experimental.pallas import tpu as pltpu
from jax.experimental.pallas import tpu_sc as plsc

N_NODES = 10000
N_EDGES = 320000
L_EDGES = 1280000
NODE_DIM = 128
EDGE_DIM = 16

NC = 2
NS = 16
NW = NC * NS

G = 128
ZROWS = 128
DCH = 512

NP = 2
SEG_S = 80128
SEG_SP = SEG_S + 16
TILE_SEG = SEG_S // NS
AGG_ROWS = NC * NP * SEG_S
L_PER_TILE = L_EDGES // NS
L_CHUNK = 3200
CLIST = E_CHUNK_MAX = 4000 + 2 * G

N_S = 5120
N_SP = N_S + 16
NTILE_SEG = N_S // NS
E_PER_TILE = N_EDGES // NS
E_CHUNK = 4000
E_CHUNK2 = 2000
E2_PER_TILE = N_EDGES // NW
G2 = 80

R128 = N_EDGES * EDGE_DIM // 128
BLK128 = 1000
NBLK = R128 // BLK128


@functools.cache
def _mesh():
    return plsc.VectorSubcoreMesh(
        core_axis_name="c", subcore_axis_name="s",
        num_cores=NC, num_subcores=NS)


_SC_PARAMS = dict(
    compiler_params=pltpu.CompilerParams(
        use_tc_tiling_on_sc=False, needs_layout_passes=False))


def _proj_body(x_ref, wt_ref, out_ref):
    out_ref[...] = 0.5 * jnp.dot(
        x_ref[...], wt_ref[...], preferred_element_type=jnp.float32)


def _proj(x, w_t):
    return pl.pallas_call(
        _proj_body,
        out_shape=jax.ShapeDtypeStruct((N_NODES, EDGE_DIM), jnp.float32),
    )(x, w_t)


def _fused_body(xp_hbm, src_hbm, dst_hbm, ea_hbm, out_hbm,
                idx_s, idx_d, rows_s, rows_d, rows_s2, rows_d2, acc,
                sem_s, sem_d, sem_s2, sem_d2):
    c = lax.axis_index("c")
    s = lax.axis_index("s")
    wid = c * NS + s
    base = wid * E2_PER_TILE
    nchunks = E2_PER_TILE // E_CHUNK2

    nbatch = E_CHUNK2 // G2
    npairs = (nbatch - 1) // 2

    def chunk(k, _):
        off = base + k * E_CHUNK2
        pltpu.sync_copy(src_hbm.at[pl.ds(off, E_CHUNK2)], idx_s)
        pltpu.sync_copy(dst_hbm.at[pl.ds(off, E_CHUNK2)], idx_d)
        pltpu.sync_copy(ea_hbm.at[pl.ds(off, E_CHUNK2)], acc)

        def fire(b, rs, rd, ss, sd):
            boff = pl.multiple_of(b * G2, 8)
            pltpu.async_copy(xp_hbm.at[idx_s.at[pl.ds(boff, G2)]], rs, ss)
            pltpu.async_copy(xp_hbm.at[idx_d.at[pl.ds(boff, G2)]], rd, sd)

        def wait(rs, rd, ss, sd):
            pltpu.make_async_copy(
                xp_hbm.at[idx_s.at[pl.ds(0, G2)]], rs, ss).wait()
            pltpu.make_async_copy(
                xp_hbm.at[idx_d.at[pl.ds(0, G2)]], rd, sd).wait()

        def add(b, rs, rd):
            boff = pl.multiple_of(b * G2, 8)

            def row(r, _):
                acc[boff + r] = acc[boff + r] + rs[r] + rd[r]
                return 0

            lax.fori_loop(0, G2, row, 0)

        fire(0, rows_s, rows_d, sem_s, sem_d)

        def pair(ip, _):
            b0 = ip * 2
            fire(b0 + 1, rows_s2, rows_d2, sem_s2, sem_d2)
            wait(rows_s, rows_d, sem_s, sem_d)
            add(b0, rows_s, rows_d)
            fire(b0 + 2, rows_s, rows_d, sem_s, sem_d)
            wait(rows_s2, rows_d2, sem_s2, sem_d2)
            add(b0 + 1, rows_s2, rows_d2)
            return 0

        lax.fori_loop(0, npairs, pair, 0)
        wait(rows_s, rows_d, sem_s, sem_d)
        add(nbatch - 1, rows_s, rows_d)
        pltpu.sync_copy(acc, out_hbm.at[pl.ds(off, E_CHUNK2)])
        return 0

    lax.fori_loop(0, nchunks, chunk, 0)


def _fused_stage(xp_half, src, dst, edge_attr):
    k = pl.kernel(
        _fused_body,
        out_type=jax.ShapeDtypeStruct((N_EDGES, EDGE_DIM), jnp.float32),
        mesh=_mesh(), **_SC_PARAMS,
        scratch_types=[
            pltpu.VMEM((E_CHUNK2,), jnp.int32),
            pltpu.VMEM((E_CHUNK2,), jnp.int32),
            pltpu.VMEM((G2, EDGE_DIM), jnp.float32),
            pltpu.VMEM((G2, EDGE_DIM), jnp.float32),
            pltpu.VMEM((G2, EDGE_DIM), jnp.float32),
            pltpu.VMEM((G2, EDGE_DIM), jnp.float32),
            pltpu.VMEM((E_CHUNK2, EDGE_DIM), jnp.float32),
            pltpu.SemaphoreType.DMA,
            pltpu.SemaphoreType.DMA,
            pltpu.SemaphoreType.DMA,
            pltpu.SemaphoreType.DMA,
        ],
    )
    return k(xp_half, src, dst, edge_attr)


def _filter_chunk(vals, keys, nvec, seg_base, seg_span, clist, dlist,
                  val_is_pos, off):
    def vec(i, cur):
        d = keys[pl.ds(i * 16, 16)]
        loc = d - seg_base
        mask = (loc >= 0) & (loc < seg_span)
        loc = jnp.where(mask, loc, seg_span)
        if val_is_pos:
            u = off + i * 16 + lax.iota(jnp.int32, 16)
        else:
            u = vals[pl.ds(i * 16, 16)]
        plsc.store_compressed(clist.at[pl.ds(cur, 16)], u, mask=mask)
        plsc.store_compressed(dlist.at[pl.ds(cur, 16)], loc, mask=mask)
        return cur + jnp.sum(mask.astype(jnp.int32))

    nc = lax.fori_loop(0, nvec, vec, 0)

    zi = jnp.zeros((16,), jnp.int32)
    di = jnp.full((16,), seg_span, jnp.int32)
    for j in range(2 * G // 16):
        clist[pl.ds(nc + j * 16, 16)] = zi
        dlist[pl.ds(nc + j * 16, 16)] = di
    return nc


def _drain_chunk(nc, table_hbm, clist, dlist, dbufA, dbufB, rowsA, rowsB,
                 ones, sums_sp, cnts_sp, semA, semB,
                 semSA, semSB, semC):
    nb2 = jnp.maximum(2 * ((nc + 2 * G - 1) // (2 * G)), 2)

    def gather(b, rows, sem):
        boff = pl.multiple_of(b * G, 8)
        return pltpu.async_copy(
            table_hbm.at[clist.at[pl.ds(boff, G)]], rows, sem)

    def wait_gather(rows, sem):
        pltpu.make_async_copy(
            table_hbm.at[clist.at[pl.ds(0, G)]], rows, sem).wait()

    def consume(b, rows, dbuf):
        boff = pl.multiple_of(b * G, 8)
        for j in range(G // 16):
            dbuf[pl.ds(j * 16, 16)] = dlist[pl.ds(boff + j * 16, 16)]
        pltpu.sync_copy(rows, sums_sp.at[dbuf], add=True)
        pltpu.sync_copy(ones, cnts_sp.at[dbuf], add=True)

    gather(0, rowsA, semA)

    def pair(ip, _):
        b0 = ip * 2
        gather(b0 + 1, rowsB, semB)
        wait_gather(rowsA, semA)
        consume(b0, rowsA, dbufA)

        @pl.when(b0 + 2 < nb2)
        def _():
            gather(b0 + 2, rowsA, semA)

        wait_gather(rowsB, semB)
        consume(b0 + 1, rowsB, dbufB)
        return 0

    lax.fori_loop(0, nb2 // 2, pair, 0)


def _zero_region(s, tile_rows, zb, zc, sums_sp, cnts_sp):
    row0 = s * tile_rows
    nfull = tile_rows // ZROWS
    for j in range(nfull):
        pltpu.sync_copy(zb, sums_sp.at[pl.ds(row0 + j * ZROWS, ZROWS)])
        pltpu.sync_copy(zc, cnts_sp.at[pl.ds(row0 + j * ZROWS, ZROWS)])
    rem = tile_rows - nfull * ZROWS
    if rem:
        pltpu.sync_copy(zb.at[pl.ds(0, rem)],
                        sums_sp.at[pl.ds(row0 + nfull * ZROWS, rem)])
        pltpu.sync_copy(zc.at[pl.ds(0, rem)],
                        cnts_sp.at[pl.ds(row0 + nfull * ZROWS, rem)])


def _divide_flush(s, tile_rows, out_base, dvb, cvb, sums_sp, cnts_sp,
                  out_hbm):
    row0 = s * tile_rows
    nfull = tile_rows // DCH

    def do(roff, n):
        pltpu.sync_copy(sums_sp.at[pl.ds(row0 + roff, n)], dvb.at[pl.ds(0, n)])
        pltpu.sync_copy(cnts_sp.at[pl.ds(row0 + roff, n)], cvb.at[pl.ds(0, n)])

        def row(r, _):
            cnt = plsc.load_gather(cvb, [lax.broadcast(r, (16,))])
            dvb[r] = dvb[r] / jnp.maximum(cnt, 1.0)
            return 0

        lax.fori_loop(0, n, row, 0)
        pltpu.sync_copy(dvb.at[pl.ds(0, n)],
                        out_hbm.at[pl.ds(out_base + row0 + roff, n)])

    for j in range(nfull):
        do(j * DCH, DCH)
    rem = tile_rows - nfull * DCH
    if rem:
        do(nfull * DCH, rem)


def _init_const(ones, zb, zc):
    one_v = jnp.ones((16,), jnp.float32)
    zero_v = jnp.zeros((16,), jnp.float32)
    for j in range(G // 16):
        ones[pl.ds(j * 16, 16)] = one_v
    for j in range(ZROWS // 16):
        zc[pl.ds(j * 16, 16)] = zero_v

    def zrow(r, _):
        zb[r] = zero_v
        return 0

    lax.fori_loop(0, ZROWS, zrow, 0)


def _line_body(fused_hbm, li_hbm, agg_hbm,
               libuf, clist, dlist, dbufA, dbufB, rowsA, rowsB,
               ones, zb, zc, dvb, cvb, sums_sp, cnts_sp, semA, semB,
               semSA, semSB, semC):
    c = lax.axis_index("c")
    s = lax.axis_index("s")
    _init_const(ones, zb, zc)
    ibase = s * L_PER_TILE

    for p in range(NP):
        sid = c * NP + p
        seg_base = sid * SEG_S
        _zero_region(s, TILE_SEG, zb, zc, sums_sp, cnts_sp)
        plsc.subcore_barrier()

        def chunk(k, _):
            off = ibase + k * L_CHUNK
            pltpu.sync_copy(li_hbm.at[:, pl.ds(off, L_CHUNK)], libuf)
            nc = _filter_chunk(libuf.at[0], libuf.at[1], L_CHUNK // 16,
                               seg_base, SEG_S, clist, dlist, False, 0)
            _drain_chunk(nc, fused_hbm, clist, dlist, dbufA, dbufB,
                         rowsA, rowsB, ones, sums_sp, cnts_sp, semA, semB,
                         semSA, semSB, semC)
            return 0

        lax.fori_loop(0, L_PER_TILE // L_CHUNK, chunk, 0)
        plsc.subcore_barrier()
        _divide_flush(s, TILE_SEG, seg_base, dvb, cvb, sums_sp, cnts_sp,
                      agg_hbm)
        plsc.subcore_barrier()


def _line_stage(fused, li):
    k = pl.kernel(
        _line_body,
        out_type=jax.ShapeDtypeStruct((AGG_ROWS, EDGE_DIM), jnp.float32),
        mesh=_mesh(), **_SC_PARAMS,
        scratch_types=[
            pltpu.VMEM((2, L_CHUNK), jnp.int32),
            pltpu.VMEM((CLIST,), jnp.int32),
            pltpu.VMEM((CLIST,), jnp.int32),
            pltpu.VMEM((G,), jnp.int32),
            pltpu.VMEM((G,), jnp.int32),
            pltpu.VMEM((G, EDGE_DIM), jnp.float32),
            pltpu.VMEM((G, EDGE_DIM), jnp.float32),
            pltpu.VMEM((G,), jnp.float32),
            pltpu.VMEM((ZROWS, EDGE_DIM), jnp.float32),
            pltpu.VMEM((ZROWS,), jnp.float32),
            pltpu.VMEM((DCH, EDGE_DIM), jnp.float32),
            pltpu.VMEM((DCH,), jnp.float32),
            pltpu.VMEM_SHARED((SEG_SP, EDGE_DIM), jnp.float32),
            pltpu.VMEM_SHARED((SEG_SP,), jnp.float32),
            pltpu.SemaphoreType.DMA,
            pltpu.SemaphoreType.DMA,
            pltpu.SemaphoreType.DMA,
            pltpu.SemaphoreType.DMA,
            pltpu.SemaphoreType.DMA,
        ],
    )
    return k(fused, li)


def _mlp_body(agg_ref, w_ref, b_ref, a_ref, pre_ref, stats_ref, acc_ref):
    i = pl.program_id(0)
    h = jnp.dot(agg_ref[...], w_ref[...], preferred_element_type=jnp.float32)
    h = h + b_ref[...]
    h = jnp.where(h >= 0.0, h, a_ref[0, 0] * h)
    pre_ref[...] = h

    @pl.when(i == 0)
    def _():
        acc_ref[...] = jnp.zeros_like(acc_ref)

    part = jnp.concatenate(
        [jnp.sum(h, axis=0, keepdims=True),
         jnp.sum(h * h, axis=0, keepdims=True)], axis=0)
    acc_ref[0:2, :] = acc_ref[0:2, :] + part

    @pl.when(i == NBLK - 1)
    def _():
        stats_ref[...] = acc_ref[...]


def _mlp_stage(agg128, w128, b128, ar):
    return pl.pallas_call(
        _mlp_body,
        grid=(NBLK,),
        in_specs=[
            pl.BlockSpec((BLK128, 128), lambda i: (i, 0)),
            pl.BlockSpec((128, 128), lambda i: (0, 0)),
            pl.BlockSpec((1, 128), lambda i: (0, 0)),
            pl.BlockSpec((1, 1), lambda i: (0, 0)),
        ],
        out_specs=[
            pl.BlockSpec((BLK128, 128), lambda i: (i, 0)),
            pl.BlockSpec((8, 128), lambda i: (0, 0)),
        ],
        out_shape=[
            jax.ShapeDtypeStruct((R128, 128), jnp.float32),
            jax.ShapeDtypeStruct((8, 128), jnp.float32),
        ],
        scratch_shapes=[pltpu.VMEM((8, 128), jnp.float32)],
    )(agg128, w128, b128, ar)


def _bn_body(pre_ref, fused_ref, sc_ref, sh_ref, out_ref):
    out_ref[...] = fused_ref[...] + pre_ref[...] * sc_ref[...] + sh_ref[...]


def _bn_stage(pre, fused128, scale128, shift128):
    return pl.pallas_call(
        _bn_body,
        grid=(NBLK,),
        in_specs=[
            pl.BlockSpec((BLK128, 128), lambda i: (i, 0)),
            pl.BlockSpec((BLK128, 128), lambda i: (i, 0)),
            pl.BlockSpec((1, 128), lambda i: (0, 0)),
            pl.BlockSpec((1, 128), lambda i: (0, 0)),
        ],
        out_specs=pl.BlockSpec((BLK128, 128), lambda i: (i, 0)),
        out_shape=jax.ShapeDtypeStruct((R128, 128), jnp.float32),
    )(pre, fused128, scale128, shift128)


def _node_body(f2_hbm, dst_hbm, out_hbm,
               didx, clist, dlist, dbufA, dbufB, rowsA, rowsB,
               ones, zb, zc, dvb, cvb, sums_sp, cnts_sp, semA, semB,
               semSA, semSB, semC):
    c = lax.axis_index("c")
    s = lax.axis_index("s")
    _init_const(ones, zb, zc)
    seg_base = c * N_S
    ibase = s * E_PER_TILE

    _zero_region(s, NTILE_SEG, zb, zc, sums_sp, cnts_sp)
    plsc.subcore_barrier()

    def chunk(k, _):
        off = ibase + k * E_CHUNK
        pltpu.sync_copy(dst_hbm.at[pl.ds(off, E_CHUNK)], didx)
        nc = _filter_chunk(None, didx, E_CHUNK // 16, seg_base, N_S,
                           clist, dlist, True, off)
        _drain_chunk(nc, f2_hbm, clist, dlist, dbufA, dbufB,
                     rowsA, rowsB, ones, sums_sp, cnts_sp, semA, semB,
                     semSA, semSB, semC)
        return 0

    lax.fori_loop(0, E_PER_TILE // E_CHUNK, chunk, 0)
    plsc.subcore_barrier()
    _divide_flush(s, NTILE_SEG, seg_base, dvb, cvb, sums_sp, cnts_sp,
                  out_hbm)


def _node_stage(fused2, dst):
    k = pl.kernel(
        _node_body,
        out_type=jax.ShapeDtypeStruct((NC * N_S, EDGE_DIM), jnp.float32),
        mesh=_mesh(), **_SC_PARAMS,
        scratch_types=[
            pltpu.VMEM((E_CHUNK,), jnp.int32),
            pltpu.VMEM((CLIST,), jnp.int32),
            pltpu.VMEM((CLIST,), jnp.int32),
            pltpu.VMEM((G,), jnp.int32),
            pltpu.VMEM((G,), jnp.int32),
            pltpu.VMEM((G, EDGE_DIM), jnp.float32),
            pltpu.VMEM((G, EDGE_DIM), jnp.float32),
            pltpu.VMEM((G,), jnp.float32),
            pltpu.VMEM((ZROWS, EDGE_DIM), jnp.float32),
            pltpu.VMEM((ZROWS,), jnp.float32),
            pltpu.VMEM((DCH, EDGE_DIM), jnp.float32),
            pltpu.VMEM((DCH,), jnp.float32),
            pltpu.VMEM_SHARED((N_SP, EDGE_DIM), jnp.float32),
            pltpu.VMEM_SHARED((N_SP,), jnp.float32),
            pltpu.SemaphoreType.DMA,
            pltpu.SemaphoreType.DMA,
            pltpu.SemaphoreType.DMA,
            pltpu.SemaphoreType.DMA,
            pltpu.SemaphoreType.DMA,
        ],
    )
    return k(fused2, dst)


def kernel(x, edge_index, edge_attr, line_graph_edge_index,
           W_proj, W1, b1, prelu_a, bn_gamma, bn_beta):
    src = edge_index[0]
    dst = edge_index[1]
    xp_half = _proj(x, W_proj.T)
    fused = _fused_stage(xp_half, src, dst, edge_attr)
    agg = _line_stage(fused, line_graph_edge_index)

    agg128 = agg.reshape(AGG_ROWS * EDGE_DIM // 128, 128)
    w128 = jax.scipy.linalg.block_diag(*([W1.T] * 8))
    b128 = jnp.tile(b1, 8).reshape(1, 128)
    pre, stats = _mlp_stage(agg128, w128, b128, prelu_a.reshape(1, 1))

    n = float(N_EDGES)
    mu = stats[0].reshape(8, EDGE_DIM).sum(axis=0) / n
    var = stats[1].reshape(8, EDGE_DIM).sum(axis=0) / n - mu * mu
    inv = lax.rsqrt(var + 1e-5)
    scale = inv * bn_gamma
    shift = bn_beta - mu * scale
    scale128 = jnp.tile(scale, 8).reshape(1, 128)
    shift128 = jnp.tile(shift, 8).reshape(1, 128)

    fused128 = fused.reshape(R128, 128)
    fused2 = _bn_stage(pre, fused128, scale128, shift128)
    out = _node_stage(fused2.reshape(N_EDGES, EDGE_DIM), dst)
    return out[:N_NODES]


# speedup vs baseline: 1.8654x; 1.8654x over previous
import functools

import jax
import jax.numpy as jnp
from jax import lax
from jax.experimental import pallas as pl
from jax.experimental.pallas import tpu as pltpu
from jax.experimental.pallas import tpu_sc as plsc

N_NODES = 10000
N_EDGES = 320000
L_EDGES = 1280000
NODE_DIM = 128
EDGE_DIM = 16

NC = 2
NS = 16
NW = NC * NS

G = 128
ZROWS = 128
DCH = 512

NP = 2
SEG_S = 80128
SEG_SP = SEG_S + 16
TILE_SEG = SEG_S // NS
AGG_ROWS = NC * NP * SEG_S
L_PER_TILE = L_EDGES // NS
L_CHUNK = 4000
CLIST = E_CHUNK_MAX = 4000 + 2 * G

N_S = 5120
N_SP = N_S + 16
NTILE_SEG = N_S // NS
E_PER_TILE = N_EDGES // NS
E_CHUNK = 4000
E_CHUNK2 = 2000
E2_PER_TILE = N_EDGES // NW
G2 = 80

R128 = N_EDGES * EDGE_DIM // 128
BLK128 = 1000
NBLK = R128 // BLK128


@functools.cache
def _mesh():
    return plsc.VectorSubcoreMesh(
        core_axis_name="c", subcore_axis_name="s",
        num_cores=NC, num_subcores=NS)


_SC_PARAMS = dict(
    compiler_params=pltpu.CompilerParams(
        use_tc_tiling_on_sc=False, needs_layout_passes=False))


def _proj_body(x_ref, wt_ref, out_ref):
    out_ref[...] = 0.5 * jnp.dot(
        x_ref[...], wt_ref[...], preferred_element_type=jnp.float32)


def _proj(x, w_t):
    return pl.pallas_call(
        _proj_body,
        out_shape=jax.ShapeDtypeStruct((N_NODES, EDGE_DIM), jnp.float32),
    )(x, w_t)


def _fused_body(xp_hbm, src_hbm, dst_hbm, ea_hbm, out_hbm,
                idx_s, idx_d, rows_s, rows_d, rows_s2, rows_d2, acc,
                sem_s, sem_d, sem_s2, sem_d2):
    c = lax.axis_index("c")
    s = lax.axis_index("s")
    wid = c * NS + s
    base = wid * E2_PER_TILE
    nchunks = E2_PER_TILE // E_CHUNK2

    nbatch = E_CHUNK2 // G2
    npairs = (nbatch - 1) // 2

    def chunk(k, _):
        off = base + k * E_CHUNK2
        pltpu.sync_copy(src_hbm.at[pl.ds(off, E_CHUNK2)], idx_s)
        pltpu.sync_copy(dst_hbm.at[pl.ds(off, E_CHUNK2)], idx_d)
        pltpu.sync_copy(ea_hbm.at[pl.ds(off, E_CHUNK2)], acc)

        def fire(b, rs, rd, ss, sd):
            boff = pl.multiple_of(b * G2, 8)
            pltpu.async_copy(xp_hbm.at[idx_s.at[pl.ds(boff, G2)]], rs, ss)
            pltpu.async_copy(xp_hbm.at[idx_d.at[pl.ds(boff, G2)]], rd, sd)

        def wait(rs, rd, ss, sd):
            pltpu.make_async_copy(
                xp_hbm.at[idx_s.at[pl.ds(0, G2)]], rs, ss).wait()
            pltpu.make_async_copy(
                xp_hbm.at[idx_d.at[pl.ds(0, G2)]], rd, sd).wait()

        def add(b, rs, rd):
            boff = pl.multiple_of(b * G2, 8)

            def row(r, _):
                acc[boff + r] = acc[boff + r] + rs[r] + rd[r]
                return 0

            lax.fori_loop(0, G2, row, 0)

        fire(0, rows_s, rows_d, sem_s, sem_d)

        def pair(ip, _):
            b0 = ip * 2
            fire(b0 + 1, rows_s2, rows_d2, sem_s2, sem_d2)
            wait(rows_s, rows_d, sem_s, sem_d)
            add(b0, rows_s, rows_d)
            fire(b0 + 2, rows_s, rows_d, sem_s, sem_d)
            wait(rows_s2, rows_d2, sem_s2, sem_d2)
            add(b0 + 1, rows_s2, rows_d2)
            return 0

        lax.fori_loop(0, npairs, pair, 0)
        wait(rows_s, rows_d, sem_s, sem_d)
        add(nbatch - 1, rows_s, rows_d)
        pltpu.sync_copy(acc, out_hbm.at[pl.ds(off, E_CHUNK2)])
        return 0

    lax.fori_loop(0, nchunks, chunk, 0)


def _fused_stage(xp_half, src, dst, edge_attr):
    k = pl.kernel(
        _fused_body,
        out_type=jax.ShapeDtypeStruct((N_EDGES, EDGE_DIM), jnp.float32),
        mesh=_mesh(), **_SC_PARAMS,
        scratch_types=[
            pltpu.VMEM((E_CHUNK2,), jnp.int32),
            pltpu.VMEM((E_CHUNK2,), jnp.int32),
            pltpu.VMEM((G2, EDGE_DIM), jnp.float32),
            pltpu.VMEM((G2, EDGE_DIM), jnp.float32),
            pltpu.VMEM((G2, EDGE_DIM), jnp.float32),
            pltpu.VMEM((G2, EDGE_DIM), jnp.float32),
            pltpu.VMEM((E_CHUNK2, EDGE_DIM), jnp.float32),
            pltpu.SemaphoreType.DMA,
            pltpu.SemaphoreType.DMA,
            pltpu.SemaphoreType.DMA,
            pltpu.SemaphoreType.DMA,
        ],
    )
    return k(xp_half, src, dst, edge_attr)


def _filter_chunk(vals, keys, nvec, seg_base, seg_span, clist, dlist,
                  val_is_pos, off):
    def vec(i, cur):
        d = keys[pl.ds(i * 16, 16)]
        loc = d - seg_base
        mask = (loc >= 0) & (loc < seg_span)
        loc = jnp.where(mask, loc, seg_span)
        if val_is_pos:
            u = off + i * 16 + lax.iota(jnp.int32, 16)
        else:
            u = vals[pl.ds(i * 16, 16)]
        plsc.store_compressed(clist.at[pl.ds(cur, 16)], u, mask=mask)
        plsc.store_compressed(dlist.at[pl.ds(cur, 16)], loc, mask=mask)
        return cur + jnp.sum(mask.astype(jnp.int32))

    nc = lax.fori_loop(0, nvec, vec, 0)

    zi = jnp.zeros((16,), jnp.int32)
    di = jnp.full((16,), seg_span, jnp.int32)
    for j in range(2 * G // 16):
        clist[pl.ds(nc + j * 16, 16)] = zi
        dlist[pl.ds(nc + j * 16, 16)] = di
    return nc


def _drain_chunk(nc, table_hbm, clist, dlist, dbufA, dbufB, rowsA, rowsB,
                 ones, sums_sp, cnts_sp, semA, semB,
                 semSA, semSB, semC):
    nb2 = jnp.maximum(2 * ((nc + 2 * G - 1) // (2 * G)), 2)

    def gather(b, rows, sem):
        boff = pl.multiple_of(b * G, 8)
        return pltpu.async_copy(
            table_hbm.at[clist.at[pl.ds(boff, G)]], rows, sem)

    def wait_gather(rows, sem):
        pltpu.make_async_copy(
            table_hbm.at[clist.at[pl.ds(0, G)]], rows, sem).wait()

    def consume(b, rows, dbuf):
        boff = pl.multiple_of(b * G, 8)
        for j in range(G // 16):
            dbuf[pl.ds(j * 16, 16)] = dlist[pl.ds(boff + j * 16, 16)]
        pltpu.sync_copy(rows, sums_sp.at[dbuf], add=True)
        pltpu.sync_copy(ones, cnts_sp.at[dbuf], add=True)

    gather(0, rowsA, semA)

    def pair(ip, _):
        b0 = ip * 2
        gather(b0 + 1, rowsB, semB)
        wait_gather(rowsA, semA)
        consume(b0, rowsA, dbufA)

        @pl.when(b0 + 2 < nb2)
        def _():
            gather(b0 + 2, rowsA, semA)

        wait_gather(rowsB, semB)
        consume(b0 + 1, rowsB, dbufB)
        return 0

    lax.fori_loop(0, nb2 // 2, pair, 0)


def _zero_region(s, tile_rows, zb, zc, sums_sp, cnts_sp):
    row0 = s * tile_rows
    nfull = tile_rows // ZROWS
    for j in range(nfull):
        pltpu.sync_copy(zb, sums_sp.at[pl.ds(row0 + j * ZROWS, ZROWS)])
        pltpu.sync_copy(zc, cnts_sp.at[pl.ds(row0 + j * ZROWS, ZROWS)])
    rem = tile_rows - nfull * ZROWS
    if rem:
        pltpu.sync_copy(zb.at[pl.ds(0, rem)],
                        sums_sp.at[pl.ds(row0 + nfull * ZROWS, rem)])
        pltpu.sync_copy(zc.at[pl.ds(0, rem)],
                        cnts_sp.at[pl.ds(row0 + nfull * ZROWS, rem)])


def _divide_flush(s, tile_rows, out_base, dvb, cvb, sums_sp, cnts_sp,
                  out_hbm):
    row0 = s * tile_rows
    nfull = tile_rows // DCH

    def do(roff, n):
        pltpu.sync_copy(sums_sp.at[pl.ds(row0 + roff, n)], dvb.at[pl.ds(0, n)])
        pltpu.sync_copy(cnts_sp.at[pl.ds(row0 + roff, n)], cvb.at[pl.ds(0, n)])

        def row(r, _):
            cnt = plsc.load_gather(cvb, [lax.broadcast(r, (16,))])
            dvb[r] = dvb[r] / jnp.maximum(cnt, 1.0)
            return 0

        lax.fori_loop(0, n, row, 0)
        pltpu.sync_copy(dvb.at[pl.ds(0, n)],
                        out_hbm.at[pl.ds(out_base + row0 + roff, n)])

    for j in range(nfull):
        do(j * DCH, DCH)
    rem = tile_rows - nfull * DCH
    if rem:
        do(nfull * DCH, rem)


def _init_const(ones, zb, zc):
    one_v = jnp.ones((16,), jnp.float32)
    zero_v = jnp.zeros((16,), jnp.float32)
    for j in range(G // 16):
        ones[pl.ds(j * 16, 16)] = one_v
    for j in range(ZROWS // 16):
        zc[pl.ds(j * 16, 16)] = zero_v

    def zrow(r, _):
        zb[r] = zero_v
        return 0

    lax.fori_loop(0, ZROWS, zrow, 0)


def _line_body(fused_hbm, lsrc_hbm, ldst_hbm, agg_hbm,
               lsrc, ldst, clist, dlist, dbufA, dbufB, rowsA, rowsB,
               ones, zb, zc, dvb, cvb, sums_sp, cnts_sp, semA, semB,
               semSA, semSB, semC):
    c = lax.axis_index("c")
    s = lax.axis_index("s")
    _init_const(ones, zb, zc)
    ibase = s * L_PER_TILE

    for p in range(NP):
        sid = c * NP + p
        seg_base = sid * SEG_S
        _zero_region(s, TILE_SEG, zb, zc, sums_sp, cnts_sp)
        plsc.subcore_barrier()

        def chunk(k, _):
            off = ibase + k * L_CHUNK
            pltpu.sync_copy(lsrc_hbm.at[pl.ds(off, L_CHUNK)], lsrc)
            pltpu.sync_copy(ldst_hbm.at[pl.ds(off, L_CHUNK)], ldst)
            nc = _filter_chunk(lsrc, ldst, L_CHUNK // 16,
                               seg_base, SEG_S, clist, dlist, False, 0)
            _drain_chunk(nc, fused_hbm, clist, dlist, dbufA, dbufB,
                         rowsA, rowsB, ones, sums_sp, cnts_sp, semA, semB,
                         semSA, semSB, semC)
            return 0

        lax.fori_loop(0, L_PER_TILE // L_CHUNK, chunk, 0)
        plsc.subcore_barrier()
        _divide_flush(s, TILE_SEG, seg_base, dvb, cvb, sums_sp, cnts_sp,
                      agg_hbm)
        plsc.subcore_barrier()


def _line_stage(fused, lsrc, ldst):
    k = pl.kernel(
        _line_body,
        out_type=jax.ShapeDtypeStruct((AGG_ROWS, EDGE_DIM), jnp.float32),
        mesh=_mesh(), **_SC_PARAMS,
        scratch_types=[
            pltpu.VMEM((L_CHUNK,), jnp.int32),
            pltpu.VMEM((L_CHUNK,), jnp.int32),
            pltpu.VMEM((CLIST,), jnp.int32),
            pltpu.VMEM((CLIST,), jnp.int32),
            pltpu.VMEM((G,), jnp.int32),
            pltpu.VMEM((G,), jnp.int32),
            pltpu.VMEM((G, EDGE_DIM), jnp.float32),
            pltpu.VMEM((G, EDGE_DIM), jnp.float32),
            pltpu.VMEM((G,), jnp.float32),
            pltpu.VMEM((ZROWS, EDGE_DIM), jnp.float32),
            pltpu.VMEM((ZROWS,), jnp.float32),
            pltpu.VMEM((DCH, EDGE_DIM), jnp.float32),
            pltpu.VMEM((DCH,), jnp.float32),
            pltpu.VMEM_SHARED((SEG_SP, EDGE_DIM), jnp.float32),
            pltpu.VMEM_SHARED((SEG_SP,), jnp.float32),
            pltpu.SemaphoreType.DMA,
            pltpu.SemaphoreType.DMA,
            pltpu.SemaphoreType.DMA,
            pltpu.SemaphoreType.DMA,
            pltpu.SemaphoreType.DMA,
        ],
    )
    return k(fused, lsrc, ldst)


def _mlp_body(agg_ref, w_ref, b_ref, a_ref, pre_ref, stats_ref, acc_ref):
    i = pl.program_id(0)
    h = jnp.dot(agg_ref[...], w_ref[...], preferred_element_type=jnp.float32)
    h = h + b_ref[...]
    h = jnp.where(h >= 0.0, h, a_ref[0, 0] * h)
    pre_ref[...] = h

    @pl.when(i == 0)
    def _():
        acc_ref[...] = jnp.zeros_like(acc_ref)

    part = jnp.concatenate(
        [jnp.sum(h, axis=0, keepdims=True),
         jnp.sum(h * h, axis=0, keepdims=True)], axis=0)
    acc_ref[0:2, :] = acc_ref[0:2, :] + part

    @pl.when(i == NBLK - 1)
    def _():
        stats_ref[...] = acc_ref[...]


def _mlp_stage(agg128, w128, b128, ar):
    return pl.pallas_call(
        _mlp_body,
        grid=(NBLK,),
        in_specs=[
            pl.BlockSpec((BLK128, 128), lambda i: (i, 0)),
            pl.BlockSpec((128, 128), lambda i: (0, 0)),
            pl.BlockSpec((1, 128), lambda i: (0, 0)),
            pl.BlockSpec((1, 1), lambda i: (0, 0)),
        ],
        out_specs=[
            pl.BlockSpec((BLK128, 128), lambda i: (i, 0)),
            pl.BlockSpec((8, 128), lambda i: (0, 0)),
        ],
        out_shape=[
            jax.ShapeDtypeStruct((R128, 128), jnp.float32),
            jax.ShapeDtypeStruct((8, 128), jnp.float32),
        ],
        scratch_shapes=[pltpu.VMEM((8, 128), jnp.float32)],
    )(agg128, w128, b128, ar)


def _bn_body(pre_ref, fused_ref, sc_ref, sh_ref, out_ref):
    out_ref[...] = fused_ref[...] + pre_ref[...] * sc_ref[...] + sh_ref[...]


def _bn_stage(pre, fused128, scale128, shift128):
    return pl.pallas_call(
        _bn_body,
        grid=(NBLK,),
        in_specs=[
            pl.BlockSpec((BLK128, 128), lambda i: (i, 0)),
            pl.BlockSpec((BLK128, 128), lambda i: (i, 0)),
            pl.BlockSpec((1, 128), lambda i: (0, 0)),
            pl.BlockSpec((1, 128), lambda i: (0, 0)),
        ],
        out_specs=pl.BlockSpec((BLK128, 128), lambda i: (i, 0)),
        out_shape=jax.ShapeDtypeStruct((R128, 128), jnp.float32),
    )(pre, fused128, scale128, shift128)


def _node_body(f2_hbm, dst_hbm, out_hbm,
               didx, clist, dlist, dbufA, dbufB, rowsA, rowsB,
               ones, zb, zc, dvb, cvb, sums_sp, cnts_sp, semA, semB,
               semSA, semSB, semC):
    c = lax.axis_index("c")
    s = lax.axis_index("s")
    _init_const(ones, zb, zc)
    seg_base = c * N_S
    ibase = s * E_PER_TILE

    _zero_region(s, NTILE_SEG, zb, zc, sums_sp, cnts_sp)
    plsc.subcore_barrier()

    def chunk(k, _):
        off = ibase + k * E_CHUNK
        pltpu.sync_copy(dst_hbm.at[pl.ds(off, E_CHUNK)], didx)
        nc = _filter_chunk(None, didx, E_CHUNK // 16, seg_base, N_S,
                           clist, dlist, True, off)
        _drain_chunk(nc, f2_hbm, clist, dlist, dbufA, dbufB,
                     rowsA, rowsB, ones, sums_sp, cnts_sp, semA, semB,
                     semSA, semSB, semC)
        return 0

    lax.fori_loop(0, E_PER_TILE // E_CHUNK, chunk, 0)
    plsc.subcore_barrier()
    _divide_flush(s, NTILE_SEG, seg_base, dvb, cvb, sums_sp, cnts_sp,
                  out_hbm)


def _node_stage(fused2, dst):
    k = pl.kernel(
        _node_body,
        out_type=jax.ShapeDtypeStruct((NC * N_S, EDGE_DIM), jnp.float32),
        mesh=_mesh(), **_SC_PARAMS,
        scratch_types=[
            pltpu.VMEM((E_CHUNK,), jnp.int32),
            pltpu.VMEM((CLIST,), jnp.int32),
            pltpu.VMEM((CLIST,), jnp.int32),
            pltpu.VMEM((G,), jnp.int32),
            pltpu.VMEM((G,), jnp.int32),
            pltpu.VMEM((G, EDGE_DIM), jnp.float32),
            pltpu.VMEM((G, EDGE_DIM), jnp.float32),
            pltpu.VMEM((G,), jnp.float32),
            pltpu.VMEM((ZROWS, EDGE_DIM), jnp.float32),
            pltpu.VMEM((ZROWS,), jnp.float32),
            pltpu.VMEM((DCH, EDGE_DIM), jnp.float32),
            pltpu.VMEM((DCH,), jnp.float32),
            pltpu.VMEM_SHARED((N_SP, EDGE_DIM), jnp.float32),
            pltpu.VMEM_SHARED((N_SP,), jnp.float32),
            pltpu.SemaphoreType.DMA,
            pltpu.SemaphoreType.DMA,
            pltpu.SemaphoreType.DMA,
            pltpu.SemaphoreType.DMA,
            pltpu.SemaphoreType.DMA,
        ],
    )
    return k(fused2, dst)


def kernel(x, edge_index, edge_attr, line_graph_edge_index,
           W_proj, W1, b1, prelu_a, bn_gamma, bn_beta):
    src = edge_index[0]
    dst = edge_index[1]
    xp_half = _proj(x, W_proj.T)
    fused = _fused_stage(xp_half, src, dst, edge_attr)
    agg = _line_stage(
        fused, line_graph_edge_index[0], line_graph_edge_index[1])

    agg128 = agg.reshape(AGG_ROWS * EDGE_DIM // 128, 128)
    w128 = jax.scipy.linalg.block_diag(*([W1.T] * 8))
    b128 = jnp.tile(b1, 8).reshape(1, 128)
    pre, stats = _mlp_stage(agg128, w128, b128, prelu_a.reshape(1, 1))

    n = float(N_EDGES)
    mu = stats[0].reshape(8, EDGE_DIM).sum(axis=0) / n
    var = stats[1].reshape(8, EDGE_DIM).sum(axis=0) / n - mu * mu
    inv = lax.rsqrt(var + 1e-5)
    scale = inv * bn_gamma
    shift = bn_beta - mu * scale
    scale128 = jnp.tile(scale, 8).reshape(1, 128)
    shift128 = jnp.tile(shift, 8).reshape(1, 128)

    fused128 = fused.reshape(R128, 128)
    fused2 = _bn_stage(pre, fused128, scale128, shift128)
    out = _node_stage(fused2.reshape(N_EDGES, EDGE_DIM), dst)
    return out[:N_NODES]
